# baseline (speedup 1.0000x reference)
import functools

import jax
import jax.numpy as jnp
from jax import lax
from jax.experimental import pallas as pl
from jax.experimental.pallas import tpu as pltpu
from jax.experimental.pallas import tpu_sc as plsc

NC = 2
NS = 16
NW = NC * NS
NL = 16

B = 4096
L = 50
D = 64
HIDDEN = 128
OUT = 64

BPW = B // NW
CHUNK_ROWS = 2
IDX_PER_GATHER = CHUNK_ROWS * L
N_CHUNKS = BPW // CHUNK_ROWS


def _sc_gather_pool(x_grouped, table):
    mesh = plsc.VectorSubcoreMesh(core_axis_name="c", subcore_axis_name="s")

    @functools.partial(
        pl.kernel,
        mesh=mesh,
        compiler_params=pltpu.CompilerParams(use_tc_tiling_on_sc=False),
        out_type=jax.ShapeDtypeStruct((B, D), jnp.float32),
        scratch_types=[
            pltpu.VMEM((N_CHUNKS, IDX_PER_GATHER), jnp.int32),
            pltpu.VMEM((IDX_PER_GATHER, D), jnp.float32),
            pltpu.VMEM((BPW, D), jnp.float32),
            pltpu.SemaphoreType.DMA,
        ],
    )
    def sc_kernel(x_hbm, table_hbm, out_hbm, idx_v, buf, acc_v, sem):
        wid = lax.axis_index("s") * NC + lax.axis_index("c")
        pltpu.sync_copy(x_hbm.at[wid], idx_v)

        def chunk_body(c, carry):
            pltpu.async_copy(table_hbm.at[idx_v.at[c]], buf, sem).wait()
            for r in range(CHUNK_ROWS):
                acc = [jnp.zeros((NL,), jnp.float32) for _ in range(D // NL)]
                for j in range(L):
                    for dd in range(D // NL):
                        acc[dd] = acc[dd] + buf[r * L + j, pl.ds(dd * NL, NL)]
                for dd in range(D // NL):
                    acc_v[c * CHUNK_ROWS + r, pl.ds(dd * NL, NL)] = acc[dd]
            return carry

        lax.fori_loop(0, N_CHUNKS, chunk_body, None)
        pltpu.sync_copy(acc_v, out_hbm.at[pl.ds(wid * BPW, BPW)])

    return sc_kernel(x_grouped, table)


def _mlp_body(pooled_ref, wh_ref, bh_ref, wo_ref, bo_ref, out_ref):
    h = jnp.tanh(
        jnp.dot(pooled_ref[...], wh_ref[...],
                preferred_element_type=jnp.float32) + bh_ref[...]
    )
    out_ref[...] = jnp.tanh(
        jnp.dot(h, wo_ref[...], preferred_element_type=jnp.float32)
        + bo_ref[...]
    )


def _tc_mlp(pooled, wh_t, bh, wo_t, bo):
    blk = 2048
    return pl.pallas_call(
        _mlp_body,
        out_shape=jax.ShapeDtypeStruct((B, OUT), jnp.float32),
        grid=(B // blk,),
        in_specs=[
            pl.BlockSpec((blk, D), lambda i: (i, 0)),
            pl.BlockSpec((D, HIDDEN), lambda i: (0, 0)),
            pl.BlockSpec((1, HIDDEN), lambda i: (0, 0)),
            pl.BlockSpec((HIDDEN, OUT), lambda i: (0, 0)),
            pl.BlockSpec((1, OUT), lambda i: (0, 0)),
        ],
        out_specs=pl.BlockSpec((blk, OUT), lambda i: (i, 0)),
    )(pooled, wh_t, bh, wo_t, bo)


def kernel(x, table, W_h, b_h, W_o, b_o):
    x_grouped = x.reshape(NW, N_CHUNKS, IDX_PER_GATHER)
    pooled = _sc_gather_pool(x_grouped, table)
    wh_t = W_h.T * (1.0 / L)
    return _tc_mlp(pooled, wh_t, b_h[None, :], W_o.T, b_o[None, :])


# speedup vs baseline: 1.0336x; 1.0336x over previous
import functools

import jax
import jax.numpy as jnp
from jax import lax
from jax.experimental import pallas as pl
from jax.experimental.pallas import tpu as pltpu
from jax.experimental.pallas import tpu_sc as plsc

NC = 2
NS = 16
NW = NC * NS
NL = 16

B = 4096
L = 50
D = 64
HIDDEN = 128
OUT = 64

BPW = B // NW
N_PAIR = BPW // 2
ND = D // NL


def _sc_gather_pool(x, table):
    mesh = plsc.VectorSubcoreMesh(core_axis_name="c", subcore_axis_name="s")

    @functools.partial(
        pl.kernel,
        mesh=mesh,
        compiler_params=pltpu.CompilerParams(use_tc_tiling_on_sc=False),
        out_type=jax.ShapeDtypeStruct((B, D), jnp.float32),
        scratch_types=[
            pltpu.VMEM((BPW, L), jnp.int32),
            pltpu.VMEM((L, D), jnp.float32),
            pltpu.VMEM((L, D), jnp.float32),
            pltpu.VMEM((BPW, D), jnp.float32),
            pltpu.SemaphoreType.DMA,
            pltpu.SemaphoreType.DMA,
        ],
    )
    def sc_kernel(x_hbm, table_hbm, out_hbm, idx_v, buf_a, buf_b, acc_v,
                  sem_a, sem_b):
        wid = lax.axis_index("s") * NC + lax.axis_index("c")
        base = wid * BPW
        pltpu.sync_copy(x_hbm.at[pl.ds(base, BPW)], idx_v)
        pltpu.async_copy(table_hbm.at[idx_v.at[0]], buf_a, sem_a)

        def acc_row(r, buf):
            a = [buf[0, pl.ds(dd * NL, NL)] for dd in range(ND)]
            for j in range(1, L):
                for dd in range(ND):
                    a[dd] = a[dd] + buf[j, pl.ds(dd * NL, NL)]
            for dd in range(ND):
                acc_v[r, pl.ds(dd * NL, NL)] = a[dd]

        def body(g, carry):
            r0 = 2 * g
            pltpu.async_copy(table_hbm.at[idx_v.at[r0 + 1]], buf_b, sem_b)
            pltpu.make_async_copy(table_hbm.at[idx_v.at[r0]], buf_a,
                                  sem_a).wait()
            acc_row(r0, buf_a)

            @pl.when(g < N_PAIR - 1)
            def _():
                pltpu.async_copy(table_hbm.at[idx_v.at[r0 + 2]], buf_a, sem_a)

            pltpu.make_async_copy(table_hbm.at[idx_v.at[r0 + 1]], buf_b,
                                  sem_b).wait()
            acc_row(r0 + 1, buf_b)
            return carry

        lax.fori_loop(0, N_PAIR, body, None)
        pltpu.sync_copy(acc_v, out_hbm.at[pl.ds(base, BPW)])

    return sc_kernel(x, table)


def _mlp_body(pooled_ref, wh_ref, bh_ref, wo_ref, bo_ref, out_ref):
    h = jnp.tanh(
        jnp.dot(pooled_ref[...], wh_ref[...],
                preferred_element_type=jnp.float32) + bh_ref[...]
    )
    out_ref[...] = jnp.tanh(
        jnp.dot(h, wo_ref[...], preferred_element_type=jnp.float32)
        + bo_ref[...]
    )


def _tc_mlp(pooled, wh_t, bh, wo_t, bo):
    blk = 2048
    return pl.pallas_call(
        _mlp_body,
        out_shape=jax.ShapeDtypeStruct((B, OUT), jnp.float32),
        grid=(B // blk,),
        in_specs=[
            pl.BlockSpec((blk, D), lambda i: (i, 0)),
            pl.BlockSpec((D, HIDDEN), lambda i: (0, 0)),
            pl.BlockSpec((1, HIDDEN), lambda i: (0, 0)),
            pl.BlockSpec((HIDDEN, OUT), lambda i: (0, 0)),
            pl.BlockSpec((1, OUT), lambda i: (0, 0)),
        ],
        out_specs=pl.BlockSpec((blk, OUT), lambda i: (i, 0)),
    )(pooled, wh_t, bh, wo_t, bo)


def kernel(x, table, W_h, b_h, W_o, b_o):
    pooled = _sc_gather_pool(x, table)
    wh_t = W_h.T * (1.0 / L)
    return _tc_mlp(pooled, wh_t, b_h[None, :], W_o.T, b_o[None, :])


# speedup vs baseline: 1.7602x; 1.7029x over previous
import functools

import jax
import jax.numpy as jnp
from jax import lax
from jax.experimental import pallas as pl
from jax.experimental.pallas import tpu as pltpu
from jax.experimental.pallas import tpu_sc as plsc

NC = 2
NS = 16
NW = NC * NS
NL = 16

B = 4096
L = 50
D = 64
HIDDEN = 128
OUT = 64
V = 1000000

BPW = B // NW
N_PAIR = BPW // 2
NH = HIDDEN // NL

BLKP = 8192
GRIDP = (V + BLKP - 1) // BLKP
VP = GRIDP * BLKP


def _proj_body(tt_ref, wh_ref, out_ref):
    out_ref[...] = lax.dot_general(
        tt_ref[...], wh_ref[...], (((0,), (0,)), ((), ())),
        preferred_element_type=jnp.float32)


def _tc_project(table_t, wh_scaled):
    return pl.pallas_call(
        _proj_body,
        out_shape=jax.ShapeDtypeStruct((VP, HIDDEN), jnp.float32),
        grid=(GRIDP,),
        in_specs=[
            pl.BlockSpec((D, BLKP), lambda i: (0, i)),
            pl.BlockSpec((D, HIDDEN), lambda i: (0, 0)),
        ],
        out_specs=pl.BlockSpec((BLKP, HIDDEN), lambda i: (i, 0)),
    )(table_t, wh_scaled)


def _sc_gather_pool(x, proj):
    mesh = plsc.VectorSubcoreMesh(core_axis_name="c", subcore_axis_name="s")

    @functools.partial(
        pl.kernel,
        mesh=mesh,
        compiler_params=pltpu.CompilerParams(use_tc_tiling_on_sc=True),
        out_type=jax.ShapeDtypeStruct((B, HIDDEN), jnp.float32),
        scratch_types=[
            pltpu.VMEM((BPW, L), jnp.int32),
            pltpu.VMEM((L, HIDDEN), jnp.float32),
            pltpu.VMEM((L, HIDDEN), jnp.float32),
            pltpu.VMEM((BPW, HIDDEN), jnp.float32),
            pltpu.SemaphoreType.DMA,
            pltpu.SemaphoreType.DMA,
        ],
    )
    def sc_kernel(x_hbm, proj_hbm, out_hbm, idx_v, buf_a, buf_b, acc_v,
                  sem_a, sem_b):
        wid = lax.axis_index("s") * NC + lax.axis_index("c")
        base = wid * BPW
        pltpu.sync_copy(x_hbm.at[pl.ds(base, BPW)], idx_v)
        pltpu.async_copy(proj_hbm.at[idx_v.at[0]], buf_a, sem_a)

        def acc_row(r, buf):
            a = [buf[0, pl.ds(dd * NL, NL)] for dd in range(NH)]
            for j in range(1, L):
                for dd in range(NH):
                    a[dd] = a[dd] + buf[j, pl.ds(dd * NL, NL)]
            for dd in range(NH):
                acc_v[r, pl.ds(dd * NL, NL)] = a[dd]

        def body(g, carry):
            r0 = 2 * g
            pltpu.async_copy(proj_hbm.at[idx_v.at[r0 + 1]], buf_b, sem_b)
            pltpu.make_async_copy(proj_hbm.at[idx_v.at[r0]], buf_a,
                                  sem_a).wait()
            acc_row(r0, buf_a)

            @pl.when(g < N_PAIR - 1)
            def _():
                pltpu.async_copy(proj_hbm.at[idx_v.at[r0 + 2]], buf_a, sem_a)

            pltpu.make_async_copy(proj_hbm.at[idx_v.at[r0 + 1]], buf_b,
                                  sem_b).wait()
            acc_row(r0 + 1, buf_b)
            return carry

        lax.fori_loop(0, N_PAIR, body, None)
        pltpu.sync_copy(acc_v, out_hbm.at[pl.ds(base, BPW)])

    return sc_kernel(x, proj)


def _mlp_body(pooled_ref, bh_ref, wo_ref, bo_ref, out_ref):
    h = jnp.tanh(pooled_ref[...] + bh_ref[...])
    out_ref[...] = jnp.tanh(
        jnp.dot(h, wo_ref[...], preferred_element_type=jnp.float32)
        + bo_ref[...]
    )


def _tc_mlp(pooled, bh, wo_t, bo):
    blk = 2048
    return pl.pallas_call(
        _mlp_body,
        out_shape=jax.ShapeDtypeStruct((B, OUT), jnp.float32),
        grid=(B // blk,),
        in_specs=[
            pl.BlockSpec((blk, HIDDEN), lambda i: (i, 0)),
            pl.BlockSpec((1, HIDDEN), lambda i: (0, 0)),
            pl.BlockSpec((HIDDEN, OUT), lambda i: (0, 0)),
            pl.BlockSpec((1, OUT), lambda i: (0, 0)),
        ],
        out_specs=pl.BlockSpec((blk, OUT), lambda i: (i, 0)),
    )(pooled, bh, wo_t, bo)


def kernel(x, table, W_h, b_h, W_o, b_o):
    wh_scaled = W_h.T * (1.0 / L)
    proj = _tc_project(table.T, wh_scaled)
    pooled = _sc_gather_pool(x, proj)
    return _tc_mlp(pooled, b_h[None, :], W_o.T, b_o[None, :])
